# baseline (speedup 1.0000x reference)
import jax
import jax.numpy as jnp
from jax import lax
from jax.experimental import pallas as pl
from jax.experimental.pallas import tpu as pltpu
from jax.experimental.pallas import tpu_sc as plsc

N_NODES = 50000
EMB = 64
HEMB = EMB // 2
EDGE_T = 16
BATCH = 1024
REGS = 1e-5

_BLK = 2000
_ACC_W = 40
_NUM0 = 8
_CHUNK = 16
_TOT_CHUNKS = N_NODES // _CHUNK
_NPAIRS = (_TOT_CHUNKS // 16 + 2) // 2
_B3 = 3 * BATCH
_PT = _B3 // 16
_PTA = _B3 // 32


def _dense_body(ae, w, asv, adv, xlo_o, xhi_o, s_o, d_o, i0_o, i1_o):
    x = ae[...]
    xp = lax.dot_general(x, w[...], (((1,), (1,)), ((), ())),
                         preferred_element_type=jnp.float32)
    xlo, xhi = xp[:, :HEMB], xp[:, HEMB:]
    xlo_o[...] = xlo
    xhi_o[...] = xhi
    s = lax.dot_general(xp, asv[...], (((1,), (1,)), ((), ())),
                        preferred_element_type=jnp.float32)
    d = lax.dot_general(xp, adv[...], (((1,), (1,)), ((), ())),
                        preferred_element_type=jnp.float32)
    s_o[...] = s
    d_o[...] = d
    a = s + d
    a = jnp.where(a >= 0, a, 0.2 * a)
    exs = jnp.exp(a)
    pad = jnp.zeros((_BLK, _NUM0 - 1), jnp.float32)
    i0_o[...] = jnp.concatenate([exs, pad, exs * xlo],
                                axis=1).astype(jnp.bfloat16)
    i1_o[...] = jnp.concatenate([exs, pad, exs * xhi],
                                axis=1).astype(jnp.bfloat16)


def _dense_phase(all_embed, W, att_src, att_dst):
    grid = N_NODES // _BLK
    outs = pl.pallas_call(
        _dense_body,
        grid=(grid,),
        in_specs=[
            pl.BlockSpec((_BLK, EMB), lambda i: (i, 0)),
            pl.BlockSpec((EMB, EMB), lambda i: (0, 0)),
            pl.BlockSpec((1, EMB), lambda i: (0, 0)),
            pl.BlockSpec((1, EMB), lambda i: (0, 0)),
        ],
        out_specs=[
            pl.BlockSpec((_BLK, HEMB), lambda i: (i, 0)),
            pl.BlockSpec((_BLK, HEMB), lambda i: (i, 0)),
            pl.BlockSpec((_BLK, 1), lambda i: (i, 0)),
            pl.BlockSpec((_BLK, 1), lambda i: (i, 0)),
            pl.BlockSpec((_BLK, _ACC_W), lambda i: (i, 0)),
            pl.BlockSpec((_BLK, _ACC_W), lambda i: (i, 0)),
        ],
        out_shape=[
            jax.ShapeDtypeStruct((N_NODES, HEMB), jnp.float32),
            jax.ShapeDtypeStruct((N_NODES, HEMB), jnp.float32),
            jax.ShapeDtypeStruct((N_NODES, 1), jnp.float32),
            jax.ShapeDtypeStruct((N_NODES, 1), jnp.float32),
            jax.ShapeDtypeStruct((N_NODES, _ACC_W), jnp.bfloat16),
            jax.ShapeDtypeStruct((N_NODES, _ACC_W), jnp.bfloat16),
        ],
    )(all_embed, W, att_src.reshape(1, EMB), att_dst.reshape(1, EMB))
    return outs


def _rnd(x):
    return lax.bitcast_convert_type(x, jnp.int32) + jnp.int32(0x8000)


def _to_bf32(a, b):
    lo = lax.shift_right_logical(_rnd(a), jnp.int32(16))
    hi = _rnd(b) & jnp.int32(-65536)
    return plsc.bitcast(lo | hi, jnp.bfloat16)


def _sc_body(edges, asrc, adst, xplo, xphi, init0, init1, bidx, ae, dmask,
             gat_o, ae_o,
             acc, eb0, eb1, xb0, xb1, ab0, ab1, advA, advB,
             pay0, pay1, pay2, pay3, pix0, pix1, pix2, pix3,
             bidx_v, aeidx, aebuf, dmv,
             sem_in, semg0, semg1, sem_sc):
    c = lax.axis_index("c")
    s = lax.axis_index("s")
    ebufs, xbufs, abufs = (eb0, eb1), (xb0, xb1), (ab0, ab1)
    pays, pixs = (pay0, pay1, pay2, pay3), (pix0, pix1, pix2, pix3)
    pltpu.sync_copy(dmask, dmv)

    @pl.when(c == 0)
    def _():
        pltpu.sync_copy(init0.at[pl.ds(s * 3120, 3120)],
                        acc.at[pl.ds(s * 3120, 3120)])

        @pl.when(s == 0)
        def _():
            pltpu.sync_copy(init0.at[pl.ds(49920, 80)],
                            acc.at[pl.ds(49920, 80)])

    @pl.when(c == 1)
    def _():
        pltpu.sync_copy(init1.at[pl.ds(s * 3120, 3120)],
                        acc.at[pl.ds(s * 3120, 3120)])

        @pl.when(s == 0)
        def _():
            pltpu.sync_copy(init1.at[pl.ds(49920, 80)],
                            acc.at[pl.ds(49920, 80)])

    plsc.subcore_barrier()

    nchunks = jnp.where(s < _TOT_CHUNKS % 16,
                        _TOT_CHUNKS // 16 + 1, _TOT_CHUNKS // 16)
    lanes = lax.iota(jnp.int32, 16)

    def issue_inputs(t, bb):
        base = (s + t * 16) * _CHUNK
        pltpu.async_copy(edges.at[pl.ds(base * EDGE_T, _CHUNK * EDGE_T)],
                         ebufs[bb], sem_in)
        pltpu.async_copy(asrc.at[pl.ds(base, _CHUNK)],
                         abufs[bb].at[pl.ds(0, _CHUNK)], sem_in)

        @pl.when(c == 0)
        def _():
            pltpu.async_copy(xplo.at[pl.ds(base, _CHUNK)], xbufs[bb], sem_in)

        @pl.when(c == 1)
        def _():
            pltpu.async_copy(xphi.at[pl.ds(base, _CHUNK)], xbufs[bb], sem_in)

    def wait_inputs(bb):
        pltpu.make_async_copy(edges.at[pl.ds(0, _CHUNK * EDGE_T)],
                              ebufs[bb], sem_in).wait()
        pltpu.make_async_copy(asrc.at[pl.ds(0, _CHUNK)],
                              abufs[bb].at[pl.ds(0, _CHUNK)], sem_in).wait()
        pltpu.make_async_copy(xplo.at[pl.ds(0, _CHUNK)],
                              xbufs[bb], sem_in).wait()

    issue_inputs(0, 0)

    def pair_body(tt, carry):
        for bb in range(2):
            t = tt * 2 + bb

            @pl.when(t < nchunks)
            def _process(t=t, bb=bb):
                base = (s + t * 16) * _CHUNK
                wait_inputs(bb)

                @pl.when(t + 1 < nchunks)
                def _():
                    issue_inputs(t + 1, 1 - bb)

                hA = pltpu.async_copy(adst.at[ebufs[bb].at[pl.ds(0, 128)]],
                                      advA, semg0)
                hB = pltpu.async_copy(adst.at[ebufs[bb].at[pl.ds(128, 128)]],
                                      advB, semg1)
                hsc = [None] * 16
                for g in range(16):
                    adv = advA if g < 8 else advB
                    if g == 0:
                        hA.wait()
                    if g == 8:
                        hB.wait()
                    pg, pp = pays[g % 4], pixs[g % 4]
                    dm = dmv[...]
                    zv = dm * 0.0
                    for k in range(1):
                        rr = g + k
                        ev = ebufs[bb][pl.ds(rr * 16, 16)]
                        ad = adv[pl.ds((rr * 16) % 128, 16)]
                        a = abufs[bb][pl.ds(rr, 16)][0] + ad
                        a = jnp.where(a >= 0, a, 0.2 * a)
                        ex = jnp.where(ev == base + rr, 0.0, jnp.exp(a))
                        pp[pl.ds(k * 16, 16)] = ev
                        x0 = xbufs[bb][rr, pl.ds(0, 16)]
                        x1 = xbufs[bb][rr, pl.ds(16, 16)]
                        for e in range(16):
                            se = ex[e]
                            row = k * 16 + e
                            pg[row, pl.ds(0, 32)] = _to_bf32(se * dm, zv)
                            pg[row, pl.ds(_NUM0, 32)] = _to_bf32(se * x0,
                                                                 se * x1)
                    if g < 3:
                        @pl.when(t > 0)
                        def _(g=g):
                            pltpu.make_async_copy(pays[g + 1],
                                                  acc.at[pixs[g + 1]],
                                                  sem_sc).wait()
                    else:
                        hsc[g - 3].wait()
                    hsc[g] = pltpu.async_copy(pg, acc.at[pp], sem_sc,
                                              add=True)

        return carry

    lax.fori_loop(0, _NPAIRS, pair_body, 0)
    for d in (1, 2, 3):
        pltpu.make_async_copy(pays[d], acc.at[pixs[d]], sem_sc).wait()
    plsc.subcore_barrier()

    j0 = s * _PT
    pltpu.sync_copy(bidx.at[pl.ds(j0, _PT)], bidx_v)
    for p in range(_PT // 16):
        pltpu.async_copy(acc.at[bidx_v.at[pl.ds(p * 16, 16)]],
                         pay0, sem_in).wait()
        pltpu.sync_copy(pay0, gat_o.at[c, pl.ds(j0 + p * 16, 16)])

    for p in range(_PTA // 16):
        a0 = c * (_B3 // 2) + s * _PTA + p * 16
        pltpu.sync_copy(bidx.at[pl.ds(a0, 16)], aeidx)
        pltpu.async_copy(ae.at[aeidx], aebuf, sem_in).wait()
        pltpu.sync_copy(aebuf, ae_o.at[pl.ds(a0, 16)])


def _sc_phase(edges_flat, a_src, a_dst, xp_lo, xp_hi, init0, init1, bidx,
              all_embed, dmask):
    mesh = plsc.VectorSubcoreMesh(core_axis_name="c", subcore_axis_name="s")
    f = pl.kernel(
        _sc_body,
        mesh=mesh,
        compiler_params=pltpu.CompilerParams(use_tc_tiling_on_sc=False,
                                             needs_layout_passes=False),
        out_type=[
            jax.ShapeDtypeStruct((2, _B3, _ACC_W), jnp.bfloat16),
            jax.ShapeDtypeStruct((_B3, EMB), jnp.float32),
        ],
        scratch_types=[
            pltpu.VMEM_SHARED((N_NODES, _ACC_W), jnp.bfloat16),
            pltpu.VMEM((_CHUNK * EDGE_T,), jnp.int32),
            pltpu.VMEM((_CHUNK * EDGE_T,), jnp.int32),
            pltpu.VMEM((_CHUNK, HEMB), jnp.float32),
            pltpu.VMEM((_CHUNK, HEMB), jnp.float32),
            pltpu.VMEM((_CHUNK + 16,), jnp.float32),
            pltpu.VMEM((_CHUNK + 16,), jnp.float32),
            pltpu.VMEM((128,), jnp.float32),
            pltpu.VMEM((128,), jnp.float32),
            pltpu.VMEM((16, _ACC_W), jnp.bfloat16),
            pltpu.VMEM((16, _ACC_W), jnp.bfloat16),
            pltpu.VMEM((16, _ACC_W), jnp.bfloat16),
            pltpu.VMEM((16, _ACC_W), jnp.bfloat16),
            pltpu.VMEM((16,), jnp.int32),
            pltpu.VMEM((16,), jnp.int32),
            pltpu.VMEM((16,), jnp.int32),
            pltpu.VMEM((16,), jnp.int32),
            pltpu.VMEM((_PT,), jnp.int32),
            pltpu.VMEM((16,), jnp.int32),
            pltpu.VMEM((16, EMB), jnp.float32),
            pltpu.VMEM((16,), jnp.float32),
            pltpu.SemaphoreType.DMA,
            pltpu.SemaphoreType.DMA,
            pltpu.SemaphoreType.DMA,
            pltpu.SemaphoreType.DMA,
        ],
    )
    return f(edges_flat, a_src, a_dst, xp_lo, xp_hi, init0, init1, bidx,
             all_embed, dmask)


def _loss_body(gat, aer, b, loss_o, bpr_o, reg_o):
    gt = gat[...].astype(jnp.float32)
    den = gt[0, :, 0:1]
    num = jnp.concatenate([gt[0, :, _NUM0:_NUM0 + HEMB],
                           gt[1, :, _NUM0:_NUM0 + HEMB]], axis=1)
    gr = num / (den + 1e-16) + b[...]
    nrm = jnp.sqrt(jnp.sum(gr * gr, axis=1, keepdims=True))
    gr = gr / jnp.clip(nrm, 1e-12, None)
    aev = aer[...]
    aeu, aep, aen = aev[:BATCH], aev[BATCH:2 * BATCH], aev[2 * BATCH:]
    gu, gp, gn = gr[:BATCH], gr[BATCH:2 * BATCH], gr[2 * BATCH:]
    ps = jnp.sum(aeu * aep + gu * gp, axis=1)
    ns = jnp.sum(aeu * aen + gu * gn, axis=1)
    bpr = -jnp.mean(jnp.log(jax.nn.sigmoid(ps - ns)))
    reg = REGS * (jnp.sum(aeu ** 2) + jnp.sum(gu ** 2)
                  + jnp.sum(aep ** 2) + jnp.sum(gp ** 2)
                  + jnp.sum(aen ** 2) + jnp.sum(gn ** 2)) * 0.5
    loss_o[...] = (bpr + reg).reshape(1, 1)
    bpr_o[...] = bpr.reshape(1, 1)
    reg_o[...] = reg.reshape(1, 1)


def _loss_phase(gat, ae_rows, bias):
    loss, bpr, reg = pl.pallas_call(
        _loss_body,
        in_specs=[
            pl.BlockSpec((2, _B3, _ACC_W), lambda: (0, 0, 0)),
            pl.BlockSpec((_B3, EMB), lambda: (0, 0)),
            pl.BlockSpec((1, EMB), lambda: (0, 0)),
        ],
        out_specs=[
            pl.BlockSpec((1, 1), lambda: (0, 0)),
            pl.BlockSpec((1, 1), lambda: (0, 0)),
            pl.BlockSpec((1, 1), lambda: (0, 0)),
        ],
        out_shape=[jax.ShapeDtypeStruct((1, 1), jnp.float32)] * 3,
    )(gat, ae_rows, bias.reshape(1, EMB))
    return loss.reshape(()), bpr.reshape(()), reg.reshape(())


def kernel(user, pos_item, neg_item, edges_matrix, all_embed, W, att_src, att_dst, bias):
    xp_lo, xp_hi, a_src, a_dst, init0, init1 = _dense_phase(
        all_embed, W, att_src, att_dst)
    bidx = jnp.concatenate([user, pos_item, neg_item]).astype(jnp.int32)
    edges_flat = edges_matrix.reshape(-1)
    dmask = jnp.zeros((16,), jnp.float32).at[0].set(1)
    gat, ae_rows = _sc_phase(edges_flat, a_src.reshape(-1), a_dst.reshape(-1),
                             xp_lo, xp_hi, init0, init1, bidx, all_embed,
                             dmask)
    loss, bpr, reg = _loss_phase(gat, ae_rows, bias)
    return (loss, bpr, reg)


# speedup vs baseline: 1.1692x; 1.1692x over previous
import jax
import jax.numpy as jnp
from jax import lax
from jax.experimental import pallas as pl
from jax.experimental.pallas import tpu as pltpu
from jax.experimental.pallas import tpu_sc as plsc

N_NODES = 50000
EMB = 64
HEMB = EMB // 2
EDGE_T = 16
BATCH = 1024
REGS = 1e-5

_BLK = 2000
_ACC_W = 36
_NUM0 = 4
_CHUNK = 16
_TOT_CHUNKS = N_NODES // _CHUNK
_NPAIRS = (_TOT_CHUNKS // 16 + 2) // 2
_B3 = 3 * BATCH
_PT = _B3 // 16
_PTA = _B3 // 32


def _dense_body(ae, w, asv, adv, xlo_o, xhi_o, s_o, d_o, i0_o, i1_o):
    x = ae[...]
    xp = lax.dot_general(x, w[...], (((1,), (1,)), ((), ())),
                         preferred_element_type=jnp.float32)
    xlo, xhi = xp[:, :HEMB], xp[:, HEMB:]
    xlo_o[...] = xlo
    xhi_o[...] = xhi
    s = lax.dot_general(xp, asv[...], (((1,), (1,)), ((), ())),
                        preferred_element_type=jnp.float32)
    d = lax.dot_general(xp, adv[...], (((1,), (1,)), ((), ())),
                        preferred_element_type=jnp.float32)
    s_o[...] = s
    d_o[...] = d
    a = s + d
    a = jnp.where(a >= 0, a, 0.2 * a)
    exs = jnp.exp(a)
    pad = jnp.zeros((_BLK, _NUM0 - 1), jnp.float32)
    i0_o[...] = jnp.concatenate([exs, pad, exs * xlo], axis=1)
    i1_o[...] = jnp.concatenate([exs, pad, exs * xhi], axis=1)


def _dense_phase(all_embed, W, att_src, att_dst):
    grid = N_NODES // _BLK
    outs = pl.pallas_call(
        _dense_body,
        grid=(grid,),
        in_specs=[
            pl.BlockSpec((_BLK, EMB), lambda i: (i, 0)),
            pl.BlockSpec((EMB, EMB), lambda i: (0, 0)),
            pl.BlockSpec((1, EMB), lambda i: (0, 0)),
            pl.BlockSpec((1, EMB), lambda i: (0, 0)),
        ],
        out_specs=[
            pl.BlockSpec((_BLK, HEMB), lambda i: (i, 0)),
            pl.BlockSpec((_BLK, HEMB), lambda i: (i, 0)),
            pl.BlockSpec((_BLK, 1), lambda i: (i, 0)),
            pl.BlockSpec((_BLK, 1), lambda i: (i, 0)),
            pl.BlockSpec((_BLK, _ACC_W), lambda i: (i, 0)),
            pl.BlockSpec((_BLK, _ACC_W), lambda i: (i, 0)),
        ],
        out_shape=[
            jax.ShapeDtypeStruct((N_NODES, HEMB), jnp.float32),
            jax.ShapeDtypeStruct((N_NODES, HEMB), jnp.float32),
            jax.ShapeDtypeStruct((N_NODES, 1), jnp.float32),
            jax.ShapeDtypeStruct((N_NODES, 1), jnp.float32),
            jax.ShapeDtypeStruct((N_NODES, _ACC_W), jnp.float32),
            jax.ShapeDtypeStruct((N_NODES, _ACC_W), jnp.float32),
        ],
    )(all_embed, W, att_src.reshape(1, EMB), att_dst.reshape(1, EMB))
    return outs


def _sc_body(edges, asrc, adst, xplo, xphi, init0, init1, bidx, ae,
             gat_o, ae_o,
             acc, eb0, eb1, xb0, xb1, ab0, ab1, advA, advB,
             pay0, pay1, pay2, pay3, pix0, pix1, pix2, pix3,
             bidx_v, aeidx, aebuf,
             sem_in, semg0, semg1, sem_sc):
    c = lax.axis_index("c")
    s = lax.axis_index("s")
    ebufs, xbufs, abufs = (eb0, eb1), (xb0, xb1), (ab0, ab1)
    pays, pixs = (pay0, pay1, pay2, pay3), (pix0, pix1, pix2, pix3)

    @pl.when(c == 0)
    def _():
        pltpu.sync_copy(init0.at[pl.ds(s * 3120, 3120)],
                        acc.at[pl.ds(s * 3120, 3120)])

        @pl.when(s == 0)
        def _():
            pltpu.sync_copy(init0.at[pl.ds(49920, 80)],
                            acc.at[pl.ds(49920, 80)])

    @pl.when(c == 1)
    def _():
        pltpu.sync_copy(init1.at[pl.ds(s * 3120, 3120)],
                        acc.at[pl.ds(s * 3120, 3120)])

        @pl.when(s == 0)
        def _():
            pltpu.sync_copy(init1.at[pl.ds(49920, 80)],
                            acc.at[pl.ds(49920, 80)])

    plsc.subcore_barrier()

    nchunks = jnp.where(s < _TOT_CHUNKS % 16,
                        _TOT_CHUNKS // 16 + 1, _TOT_CHUNKS // 16)
    lanes = lax.iota(jnp.int32, 16)

    def issue_inputs(t, bb):
        base = (s + t * 16) * _CHUNK
        pltpu.async_copy(edges.at[pl.ds(base * EDGE_T, _CHUNK * EDGE_T)],
                         ebufs[bb], sem_in)
        pltpu.async_copy(asrc.at[pl.ds(base, _CHUNK)],
                         abufs[bb].at[pl.ds(0, _CHUNK)], sem_in)

        @pl.when(c == 0)
        def _():
            pltpu.async_copy(xplo.at[pl.ds(base, _CHUNK)], xbufs[bb], sem_in)

        @pl.when(c == 1)
        def _():
            pltpu.async_copy(xphi.at[pl.ds(base, _CHUNK)], xbufs[bb], sem_in)

    def wait_inputs(bb):
        pltpu.make_async_copy(edges.at[pl.ds(0, _CHUNK * EDGE_T)],
                              ebufs[bb], sem_in).wait()
        pltpu.make_async_copy(asrc.at[pl.ds(0, _CHUNK)],
                              abufs[bb].at[pl.ds(0, _CHUNK)], sem_in).wait()
        pltpu.make_async_copy(xplo.at[pl.ds(0, _CHUNK)],
                              xbufs[bb], sem_in).wait()

    issue_inputs(0, 0)

    def pair_body(tt, carry):
        for bb in range(2):
            t = tt * 2 + bb

            @pl.when(t < nchunks)
            def _process(t=t, bb=bb):
                base = (s + t * 16) * _CHUNK
                wait_inputs(bb)

                @pl.when(t + 1 < nchunks)
                def _():
                    issue_inputs(t + 1, 1 - bb)

                hA = pltpu.async_copy(adst.at[ebufs[bb].at[pl.ds(0, 128)]],
                                      advA, semg0)
                hB = pltpu.async_copy(adst.at[ebufs[bb].at[pl.ds(128, 128)]],
                                      advB, semg1)
                hsc = [None] * 16
                for g in range(16):
                    adv = advA if g < 8 else advB
                    if g == 0:
                        hA.wait()
                    if g == 8:
                        hB.wait()
                    pg, pp = pays[g % 4], pixs[g % 4]
                    for k in range(1):
                        rr = g + k
                        ev = ebufs[bb][pl.ds(rr * 16, 16)]
                        ad = adv[pl.ds((rr * 16) % 128, 16)]
                        a = abufs[bb][pl.ds(rr, 16)][0] + ad
                        a = jnp.where(a >= 0, a, 0.2 * a)
                        ex = jnp.where(ev == base + rr, 0.0, jnp.exp(a))
                        pp[pl.ds(k * 16, 16)] = ev
                        x0 = xbufs[bb][rr, pl.ds(0, 16)]
                        x1 = xbufs[bb][rr, pl.ds(16, 16)]
                        for e in range(16):
                            se = ex[e]
                            row = k * 16 + e
                            pg[row, pl.ds(0, 16)] = jnp.where(lanes == 0,
                                                              se, 0.0)
                            pg[row, pl.ds(_NUM0, 16)] = se * x0
                            pg[row, pl.ds(_NUM0 + 16, 16)] = se * x1
                    if g < 3:
                        @pl.when(t > 0)
                        def _(g=g):
                            pltpu.make_async_copy(pays[g + 1],
                                                  acc.at[pixs[g + 1]],
                                                  sem_sc).wait()
                    else:
                        hsc[g - 3].wait()
                    hsc[g] = pltpu.async_copy(pg, acc.at[pp], sem_sc,
                                              add=True)

        return carry

    lax.fori_loop(0, _NPAIRS, pair_body, 0)
    for d in (1, 2, 3):
        pltpu.make_async_copy(pays[d], acc.at[pixs[d]], sem_sc).wait()
    plsc.subcore_barrier()

    j0 = s * _PT
    pltpu.sync_copy(bidx.at[pl.ds(j0, _PT)], bidx_v)
    for p in range(_PT // 16):
        pltpu.async_copy(acc.at[bidx_v.at[pl.ds(p * 16, 16)]],
                         pay0, sem_in).wait()
        pltpu.sync_copy(pay0, gat_o.at[c, pl.ds(j0 + p * 16, 16)])

    for p in range(_PTA // 16):
        a0 = c * (_B3 // 2) + s * _PTA + p * 16
        pltpu.sync_copy(bidx.at[pl.ds(a0, 16)], aeidx)
        pltpu.async_copy(ae.at[aeidx], aebuf, sem_in).wait()
        pltpu.sync_copy(aebuf, ae_o.at[pl.ds(a0, 16)])


def _sc_phase(edges_flat, a_src, a_dst, xp_lo, xp_hi, init0, init1, bidx,
              all_embed):
    mesh = plsc.VectorSubcoreMesh(core_axis_name="c", subcore_axis_name="s")
    f = pl.kernel(
        _sc_body,
        mesh=mesh,
        compiler_params=pltpu.CompilerParams(use_tc_tiling_on_sc=False),
        out_type=[
            jax.ShapeDtypeStruct((2, _B3, _ACC_W), jnp.float32),
            jax.ShapeDtypeStruct((_B3, EMB), jnp.float32),
        ],
        scratch_types=[
            pltpu.VMEM_SHARED((N_NODES, _ACC_W), jnp.float32),
            pltpu.VMEM((_CHUNK * EDGE_T,), jnp.int32),
            pltpu.VMEM((_CHUNK * EDGE_T,), jnp.int32),
            pltpu.VMEM((_CHUNK, HEMB), jnp.float32),
            pltpu.VMEM((_CHUNK, HEMB), jnp.float32),
            pltpu.VMEM((_CHUNK + 16,), jnp.float32),
            pltpu.VMEM((_CHUNK + 16,), jnp.float32),
            pltpu.VMEM((128,), jnp.float32),
            pltpu.VMEM((128,), jnp.float32),
            pltpu.VMEM((16, _ACC_W), jnp.float32),
            pltpu.VMEM((16, _ACC_W), jnp.float32),
            pltpu.VMEM((16, _ACC_W), jnp.float32),
            pltpu.VMEM((16, _ACC_W), jnp.float32),
            pltpu.VMEM((16,), jnp.int32),
            pltpu.VMEM((16,), jnp.int32),
            pltpu.VMEM((16,), jnp.int32),
            pltpu.VMEM((16,), jnp.int32),
            pltpu.VMEM((_PT,), jnp.int32),
            pltpu.VMEM((16,), jnp.int32),
            pltpu.VMEM((16, EMB), jnp.float32),
            pltpu.SemaphoreType.DMA,
            pltpu.SemaphoreType.DMA,
            pltpu.SemaphoreType.DMA,
            pltpu.SemaphoreType.DMA,
        ],
    )
    return f(edges_flat, a_src, a_dst, xp_lo, xp_hi, init0, init1, bidx,
             all_embed)


def _loss_body(gat, aer, b, loss_o, bpr_o, reg_o):
    gt = gat[...]
    den = gt[0, :, 0:1]
    num = jnp.concatenate([gt[0, :, _NUM0:_NUM0 + HEMB],
                           gt[1, :, _NUM0:_NUM0 + HEMB]], axis=1)
    gr = num / (den + 1e-16) + b[...]
    nrm = jnp.sqrt(jnp.sum(gr * gr, axis=1, keepdims=True))
    gr = gr / jnp.clip(nrm, 1e-12, None)
    aev = aer[...]
    aeu, aep, aen = aev[:BATCH], aev[BATCH:2 * BATCH], aev[2 * BATCH:]
    gu, gp, gn = gr[:BATCH], gr[BATCH:2 * BATCH], gr[2 * BATCH:]
    ps = jnp.sum(aeu * aep + gu * gp, axis=1)
    ns = jnp.sum(aeu * aen + gu * gn, axis=1)
    bpr = -jnp.mean(jnp.log(jax.nn.sigmoid(ps - ns)))
    reg = REGS * (jnp.sum(aeu ** 2) + jnp.sum(gu ** 2)
                  + jnp.sum(aep ** 2) + jnp.sum(gp ** 2)
                  + jnp.sum(aen ** 2) + jnp.sum(gn ** 2)) * 0.5
    loss_o[...] = (bpr + reg).reshape(1, 1)
    bpr_o[...] = bpr.reshape(1, 1)
    reg_o[...] = reg.reshape(1, 1)


def _loss_phase(gat, ae_rows, bias):
    loss, bpr, reg = pl.pallas_call(
        _loss_body,
        in_specs=[
            pl.BlockSpec((2, _B3, _ACC_W), lambda: (0, 0, 0)),
            pl.BlockSpec((_B3, EMB), lambda: (0, 0)),
            pl.BlockSpec((1, EMB), lambda: (0, 0)),
        ],
        out_specs=[
            pl.BlockSpec((1, 1), lambda: (0, 0)),
            pl.BlockSpec((1, 1), lambda: (0, 0)),
            pl.BlockSpec((1, 1), lambda: (0, 0)),
        ],
        out_shape=[jax.ShapeDtypeStruct((1, 1), jnp.float32)] * 3,
    )(gat, ae_rows, bias.reshape(1, EMB))
    return loss.reshape(()), bpr.reshape(()), reg.reshape(())


def kernel(user, pos_item, neg_item, edges_matrix, all_embed, W, att_src, att_dst, bias):
    xp_lo, xp_hi, a_src, a_dst, init0, init1 = _dense_phase(
        all_embed, W, att_src, att_dst)
    bidx = jnp.concatenate([user, pos_item, neg_item]).astype(jnp.int32)
    edges_flat = edges_matrix.reshape(-1)
    gat, ae_rows = _sc_phase(edges_flat, a_src.reshape(-1), a_dst.reshape(-1),
                             xp_lo, xp_hi, init0, init1, bidx, all_embed)
    loss, bpr, reg = _loss_phase(gat, ae_rows, bias)
    return (loss, bpr, reg)


# speedup vs baseline: 1.5121x; 1.2933x over previous
import jax
import jax.numpy as jnp
from jax import lax
from jax.experimental import pallas as pl
from jax.experimental.pallas import tpu as pltpu
from jax.experimental.pallas import tpu_sc as plsc

N_NODES = 50000
EMB = 64
HEMB = EMB // 2
EDGE_T = 16
BATCH = 1024
REGS = 1e-5

_BLK = 2000
_ACC_W = 40
_NUM0 = 8
_CHUNK = 16
_TOT_CHUNKS = N_NODES // _CHUNK
_NPAIRS = (_TOT_CHUNKS // 16 + 2) // 2
_B3 = 3 * BATCH
_PT = _B3 // 16
_PTA = _B3 // 32


def _dense_body(ae, w, asv, adv, xlo_o, xhi_o, s_o, d_o, i0_o, i1_o):
    x = ae[...]
    xp = lax.dot_general(x, w[...], (((1,), (1,)), ((), ())),
                         preferred_element_type=jnp.float32)
    xlo, xhi = xp[:, :HEMB], xp[:, HEMB:]
    xlo_o[...] = xlo
    xhi_o[...] = xhi
    s = lax.dot_general(xp, asv[...], (((1,), (1,)), ((), ())),
                        preferred_element_type=jnp.float32)
    d = lax.dot_general(xp, adv[...], (((1,), (1,)), ((), ())),
                        preferred_element_type=jnp.float32)
    s_o[...] = s
    d_o[...] = d
    a = s + d
    a = jnp.where(a >= 0, a, 0.2 * a)
    exs = jnp.exp(a)
    pad = jnp.zeros((_BLK, _NUM0 - 1), jnp.float32)
    i0_o[...] = jnp.concatenate([exs, pad, exs * xlo], axis=1)
    i1_o[...] = jnp.concatenate([exs, pad, exs * xhi], axis=1)


def _dense_phase(all_embed, W, att_src, att_dst):
    grid = N_NODES // _BLK
    outs = pl.pallas_call(
        _dense_body,
        grid=(grid,),
        in_specs=[
            pl.BlockSpec((_BLK, EMB), lambda i: (i, 0)),
            pl.BlockSpec((EMB, EMB), lambda i: (0, 0)),
            pl.BlockSpec((1, EMB), lambda i: (0, 0)),
            pl.BlockSpec((1, EMB), lambda i: (0, 0)),
        ],
        out_specs=[
            pl.BlockSpec((_BLK, HEMB), lambda i: (i, 0)),
            pl.BlockSpec((_BLK, HEMB), lambda i: (i, 0)),
            pl.BlockSpec((_BLK, 1), lambda i: (i, 0)),
            pl.BlockSpec((_BLK, 1), lambda i: (i, 0)),
            pl.BlockSpec((_BLK, _ACC_W), lambda i: (i, 0)),
            pl.BlockSpec((_BLK, _ACC_W), lambda i: (i, 0)),
        ],
        out_shape=[
            jax.ShapeDtypeStruct((N_NODES, HEMB), jnp.float32),
            jax.ShapeDtypeStruct((N_NODES, HEMB), jnp.float32),
            jax.ShapeDtypeStruct((N_NODES, 1), jnp.float32),
            jax.ShapeDtypeStruct((N_NODES, 1), jnp.float32),
            jax.ShapeDtypeStruct((N_NODES, _ACC_W), jnp.float32),
            jax.ShapeDtypeStruct((N_NODES, _ACC_W), jnp.float32),
        ],
    )(all_embed, W, att_src.reshape(1, EMB), att_dst.reshape(1, EMB))
    return outs


def _sc_body(edges, asrc, adst, xplo, xphi, init0, init1, bidx, ae,
             gat_o, ae_o,
             acc, eb0, eb1, xb0, xb1, ab0, ab1, advA0, advA1, advB0, advB1,
             pay0, pay1, pay2, pay3, pix0, pix1, pix2, pix3,
             bidx_v, aeidx, aebuf,
             sem_in, semg0, semg1, sem_sc):
    c = lax.axis_index("c")
    s = lax.axis_index("s")
    ebufs, xbufs, abufs = (eb0, eb1), (xb0, xb1), (ab0, ab1)
    advAr, advBr = (advA0, advA1), (advB0, advB1)
    pays, pixs = (pay0, pay1, pay2, pay3), (pix0, pix1, pix2, pix3)

    @pl.when(c == 0)
    def _():
        pltpu.sync_copy(init0.at[pl.ds(s * 3120, 3120)],
                        acc.at[pl.ds(s * 3120, 3120)])

        @pl.when(s == 0)
        def _():
            pltpu.sync_copy(init0.at[pl.ds(49920, 80)],
                            acc.at[pl.ds(49920, 80)])

    @pl.when(c == 1)
    def _():
        pltpu.sync_copy(init1.at[pl.ds(s * 3120, 3120)],
                        acc.at[pl.ds(s * 3120, 3120)])

        @pl.when(s == 0)
        def _():
            pltpu.sync_copy(init1.at[pl.ds(49920, 80)],
                            acc.at[pl.ds(49920, 80)])

    plsc.subcore_barrier()

    nchunks = jnp.where(s < _TOT_CHUNKS % 16,
                        _TOT_CHUNKS // 16 + 1, _TOT_CHUNKS // 16)
    lanes = lax.iota(jnp.int32, 16)

    def issue_inputs(t, bb):
        base = (s + t * 16) * _CHUNK
        pltpu.async_copy(edges.at[pl.ds(base * EDGE_T, _CHUNK * EDGE_T)],
                         ebufs[bb], sem_in)
        pltpu.async_copy(asrc.at[pl.ds(base, _CHUNK)],
                         abufs[bb].at[pl.ds(0, _CHUNK)], sem_in)

        @pl.when(c == 0)
        def _():
            pltpu.async_copy(xplo.at[pl.ds(base, _CHUNK)], xbufs[bb], sem_in)

        @pl.when(c == 1)
        def _():
            pltpu.async_copy(xphi.at[pl.ds(base, _CHUNK)], xbufs[bb], sem_in)

    def wait_inputs(bb):
        pltpu.make_async_copy(edges.at[pl.ds(0, _CHUNK * EDGE_T)],
                              ebufs[bb], sem_in).wait()
        pltpu.make_async_copy(asrc.at[pl.ds(0, _CHUNK)],
                              abufs[bb].at[pl.ds(0, _CHUNK)], sem_in).wait()
        pltpu.make_async_copy(xplo.at[pl.ds(0, _CHUNK)],
                              xbufs[bb], sem_in).wait()

    def issue_gathers(bb):
        pltpu.async_copy(adst.at[ebufs[bb].at[pl.ds(0, 128)]],
                         advAr[bb], semg0)
        pltpu.async_copy(adst.at[ebufs[bb].at[pl.ds(128, 128)]],
                         advBr[bb], semg1)

    def wait_gathers(bb, which):
        if which == 0:
            pltpu.make_async_copy(adst.at[ebufs[bb].at[pl.ds(0, 128)]],
                                  advAr[bb], semg0).wait()
        else:
            pltpu.make_async_copy(adst.at[ebufs[bb].at[pl.ds(128, 128)]],
                                  advBr[bb], semg1).wait()

    issue_inputs(0, 0)
    wait_inputs(0)
    issue_gathers(0)

    def pair_body(tt, carry):
        for bb in range(2):
            t = tt * 2 + bb

            @pl.when(t < nchunks)
            def _process(t=t, bb=bb):
                base = (s + t * 16) * _CHUNK

                @pl.when(t + 1 < nchunks)
                def _():
                    issue_inputs(t + 1, 1 - bb)

                hsc = [None] * 16
                for g in range(16):
                    adv = advAr[bb] if g < 8 else advBr[bb]
                    if g == 0:
                        wait_gathers(bb, 0)
                    if g == 8:
                        wait_gathers(bb, 1)
                        @pl.when(t + 1 < nchunks)
                        def _():
                            wait_inputs(1 - bb)
                            issue_gathers(1 - bb)
                    pg, pp = pays[g % 4], pixs[g % 4]
                    for k in range(1):
                        rr = g + k
                        ev = ebufs[bb][pl.ds(rr * 16, 16)]
                        ad = adv[pl.ds((rr * 16) % 128, 16)]
                        a = abufs[bb][pl.ds(rr, 16)][0] + ad
                        a = jnp.where(a >= 0, a, 0.2 * a)
                        ex = jnp.where(ev == base + rr, 0.0, jnp.exp(a))
                        pp[pl.ds(k * 16, 16)] = ev
                        x0 = xbufs[bb][rr, pl.ds(0, 16)]
                        x1 = xbufs[bb][rr, pl.ds(16, 16)]
                        for e in range(16):
                            se = ex[e]
                            row = k * 16 + e
                            pg[row, pl.ds(0, 16)] = jnp.where(lanes == 0,
                                                              se, 0.0)
                            pg[row, pl.ds(_NUM0, 16)] = se * x0
                            pg[row, pl.ds(_NUM0 + 16, 16)] = se * x1
                    if g < 3:
                        @pl.when(t > 0)
                        def _(g=g):
                            pltpu.make_async_copy(pays[g + 1],
                                                  acc.at[pixs[g + 1]],
                                                  sem_sc).wait()
                    else:
                        hsc[g - 3].wait()
                    hsc[g] = pltpu.async_copy(pg, acc.at[pp], sem_sc,
                                              add=True)

        return carry

    lax.fori_loop(0, _NPAIRS, pair_body, 0)
    for d in (1, 2, 3):
        pltpu.make_async_copy(pays[d], acc.at[pixs[d]], sem_sc).wait()
    plsc.subcore_barrier()

    j0 = s * _PT
    pltpu.sync_copy(bidx.at[pl.ds(j0, _PT)], bidx_v)
    for p in range(_PT // 16):
        pltpu.async_copy(acc.at[bidx_v.at[pl.ds(p * 16, 16)]],
                         pay0, sem_in).wait()
        pltpu.sync_copy(pay0, gat_o.at[c, pl.ds(j0 + p * 16, 16)])

    for p in range(_PTA // 16):
        a0 = c * (_B3 // 2) + s * _PTA + p * 16
        pltpu.sync_copy(bidx.at[pl.ds(a0, 16)], aeidx)
        pltpu.async_copy(ae.at[aeidx], aebuf, sem_in).wait()
        pltpu.sync_copy(aebuf, ae_o.at[pl.ds(a0, 16)])


def _sc_phase(edges_flat, a_src, a_dst, xp_lo, xp_hi, init0, init1, bidx,
              all_embed):
    mesh = plsc.VectorSubcoreMesh(core_axis_name="c", subcore_axis_name="s")
    f = pl.kernel(
        _sc_body,
        mesh=mesh,
        compiler_params=pltpu.CompilerParams(use_tc_tiling_on_sc=False),
        out_type=[
            jax.ShapeDtypeStruct((2, _B3, _ACC_W), jnp.float32),
            jax.ShapeDtypeStruct((_B3, EMB), jnp.float32),
        ],
        scratch_types=[
            pltpu.VMEM_SHARED((N_NODES, _ACC_W), jnp.float32),
            pltpu.VMEM((_CHUNK * EDGE_T,), jnp.int32),
            pltpu.VMEM((_CHUNK * EDGE_T,), jnp.int32),
            pltpu.VMEM((_CHUNK, HEMB), jnp.float32),
            pltpu.VMEM((_CHUNK, HEMB), jnp.float32),
            pltpu.VMEM((_CHUNK + 16,), jnp.float32),
            pltpu.VMEM((_CHUNK + 16,), jnp.float32),
            pltpu.VMEM((128,), jnp.float32),
            pltpu.VMEM((128,), jnp.float32),
            pltpu.VMEM((128,), jnp.float32),
            pltpu.VMEM((128,), jnp.float32),
            pltpu.VMEM((16, _ACC_W), jnp.float32),
            pltpu.VMEM((16, _ACC_W), jnp.float32),
            pltpu.VMEM((16, _ACC_W), jnp.float32),
            pltpu.VMEM((16, _ACC_W), jnp.float32),
            pltpu.VMEM((16,), jnp.int32),
            pltpu.VMEM((16,), jnp.int32),
            pltpu.VMEM((16,), jnp.int32),
            pltpu.VMEM((16,), jnp.int32),
            pltpu.VMEM((_PT,), jnp.int32),
            pltpu.VMEM((16,), jnp.int32),
            pltpu.VMEM((16, EMB), jnp.float32),
            pltpu.SemaphoreType.DMA,
            pltpu.SemaphoreType.DMA,
            pltpu.SemaphoreType.DMA,
            pltpu.SemaphoreType.DMA,
        ],
    )
    return f(edges_flat, a_src, a_dst, xp_lo, xp_hi, init0, init1, bidx,
             all_embed)


def _loss_body(gat, aer, b, loss_o, bpr_o, reg_o):
    gt = gat[...]
    den = gt[0, :, 0:1]
    num = jnp.concatenate([gt[0, :, _NUM0:_NUM0 + HEMB],
                           gt[1, :, _NUM0:_NUM0 + HEMB]], axis=1)
    gr = num / (den + 1e-16) + b[...]
    nrm = jnp.sqrt(jnp.sum(gr * gr, axis=1, keepdims=True))
    gr = gr / jnp.clip(nrm, 1e-12, None)
    aev = aer[...]
    aeu, aep, aen = aev[:BATCH], aev[BATCH:2 * BATCH], aev[2 * BATCH:]
    gu, gp, gn = gr[:BATCH], gr[BATCH:2 * BATCH], gr[2 * BATCH:]
    ps = jnp.sum(aeu * aep + gu * gp, axis=1)
    ns = jnp.sum(aeu * aen + gu * gn, axis=1)
    bpr = -jnp.mean(jnp.log(jax.nn.sigmoid(ps - ns)))
    reg = REGS * (jnp.sum(aeu ** 2) + jnp.sum(gu ** 2)
                  + jnp.sum(aep ** 2) + jnp.sum(gp ** 2)
                  + jnp.sum(aen ** 2) + jnp.sum(gn ** 2)) * 0.5
    loss_o[...] = (bpr + reg).reshape(1, 1)
    bpr_o[...] = bpr.reshape(1, 1)
    reg_o[...] = reg.reshape(1, 1)


def _loss_phase(gat, ae_rows, bias):
    loss, bpr, reg = pl.pallas_call(
        _loss_body,
        in_specs=[
            pl.BlockSpec((2, _B3, _ACC_W), lambda: (0, 0, 0)),
            pl.BlockSpec((_B3, EMB), lambda: (0, 0)),
            pl.BlockSpec((1, EMB), lambda: (0, 0)),
        ],
        out_specs=[
            pl.BlockSpec((1, 1), lambda: (0, 0)),
            pl.BlockSpec((1, 1), lambda: (0, 0)),
            pl.BlockSpec((1, 1), lambda: (0, 0)),
        ],
        out_shape=[jax.ShapeDtypeStruct((1, 1), jnp.float32)] * 3,
    )(gat, ae_rows, bias.reshape(1, EMB))
    return loss.reshape(()), bpr.reshape(()), reg.reshape(())


def kernel(user, pos_item, neg_item, edges_matrix, all_embed, W, att_src, att_dst, bias):
    xp_lo, xp_hi, a_src, a_dst, init0, init1 = _dense_phase(
        all_embed, W, att_src, att_dst)
    bidx = jnp.concatenate([user, pos_item, neg_item]).astype(jnp.int32)
    edges_flat = edges_matrix.reshape(-1)
    gat, ae_rows = _sc_phase(edges_flat, a_src.reshape(-1), a_dst.reshape(-1),
                             xp_lo, xp_hi, init0, init1, bidx, all_embed)
    loss, bpr, reg = _loss_phase(gat, ae_rows, bias)
    return (loss, bpr, reg)


# speedup vs baseline: 1.5176x; 1.0037x over previous
import jax
import jax.numpy as jnp
from jax import lax
from jax.experimental import pallas as pl
from jax.experimental.pallas import tpu as pltpu
from jax.experimental.pallas import tpu_sc as plsc

N_NODES = 50000
EMB = 64
HEMB = EMB // 2
EDGE_T = 16
BATCH = 1024
REGS = 1e-5

_BLK = 5000
_ACC_W = 40
_NUM0 = 8
_CHUNK = 16
_TOT_CHUNKS = N_NODES // _CHUNK
_NPAIRS = (_TOT_CHUNKS // 16 + 2) // 2
_B3 = 3 * BATCH
_PT = _B3 // 16
_PTA = _B3 // 32


def _dense_body(ae, w, asv, adv, xlo_o, xhi_o, s_o, d_o, i0_o, i1_o):
    x = ae[...]
    xp = lax.dot_general(x, w[...], (((1,), (1,)), ((), ())),
                         preferred_element_type=jnp.float32)
    xlo, xhi = xp[:, :HEMB], xp[:, HEMB:]
    xlo_o[...] = xlo
    xhi_o[...] = xhi
    s = lax.dot_general(xp, asv[...], (((1,), (1,)), ((), ())),
                        preferred_element_type=jnp.float32)
    d = lax.dot_general(xp, adv[...], (((1,), (1,)), ((), ())),
                        preferred_element_type=jnp.float32)
    s_o[...] = s
    d_o[...] = d
    a = s + d
    a = jnp.where(a >= 0, a, 0.2 * a)
    exs = jnp.exp(a)
    pad = jnp.zeros((_BLK, _NUM0 - 1), jnp.float32)
    i0_o[...] = jnp.concatenate([exs, pad, exs * xlo], axis=1)
    i1_o[...] = jnp.concatenate([exs, pad, exs * xhi], axis=1)


def _dense_phase(all_embed, W, att_src, att_dst):
    grid = N_NODES // _BLK
    outs = pl.pallas_call(
        _dense_body,
        grid=(grid,),
        in_specs=[
            pl.BlockSpec((_BLK, EMB), lambda i: (i, 0)),
            pl.BlockSpec((EMB, EMB), lambda i: (0, 0)),
            pl.BlockSpec((1, EMB), lambda i: (0, 0)),
            pl.BlockSpec((1, EMB), lambda i: (0, 0)),
        ],
        out_specs=[
            pl.BlockSpec((_BLK, HEMB), lambda i: (i, 0)),
            pl.BlockSpec((_BLK, HEMB), lambda i: (i, 0)),
            pl.BlockSpec((_BLK, 1), lambda i: (i, 0)),
            pl.BlockSpec((_BLK, 1), lambda i: (i, 0)),
            pl.BlockSpec((_BLK, _ACC_W), lambda i: (i, 0)),
            pl.BlockSpec((_BLK, _ACC_W), lambda i: (i, 0)),
        ],
        out_shape=[
            jax.ShapeDtypeStruct((N_NODES, HEMB), jnp.float32),
            jax.ShapeDtypeStruct((N_NODES, HEMB), jnp.float32),
            jax.ShapeDtypeStruct((N_NODES, 1), jnp.float32),
            jax.ShapeDtypeStruct((N_NODES, 1), jnp.float32),
            jax.ShapeDtypeStruct((N_NODES, _ACC_W), jnp.float32),
            jax.ShapeDtypeStruct((N_NODES, _ACC_W), jnp.float32),
        ],
    )(all_embed, W, att_src.reshape(1, EMB), att_dst.reshape(1, EMB))
    return outs


def _sc_body(edges, asrc, adst, xplo, xphi, init0, init1, bidx, ae,
             gat_o, ae_o,
             acc, eb0, eb1, xb0, xb1, ab0, ab1, advA0, advA1, advB0, advB1,
             pay0, pay1, pay2, pay3, pix0, pix1, pix2, pix3,
             bidx_v, aeidx, aebuf,
             sem_in, semg0, semg1, sem_sc):
    c = lax.axis_index("c")
    s = lax.axis_index("s")
    ebufs, xbufs, abufs = (eb0, eb1), (xb0, xb1), (ab0, ab1)
    advAr, advBr = (advA0, advA1), (advB0, advB1)
    pays, pixs = (pay0, pay1, pay2, pay3), (pix0, pix1, pix2, pix3)

    @pl.when(c == 0)
    def _():
        pltpu.sync_copy(init0.at[pl.ds(s * 3120, 3120)],
                        acc.at[pl.ds(s * 3120, 3120)])

        @pl.when(s == 0)
        def _():
            pltpu.sync_copy(init0.at[pl.ds(49920, 80)],
                            acc.at[pl.ds(49920, 80)])

    @pl.when(c == 1)
    def _():
        pltpu.sync_copy(init1.at[pl.ds(s * 3120, 3120)],
                        acc.at[pl.ds(s * 3120, 3120)])

        @pl.when(s == 0)
        def _():
            pltpu.sync_copy(init1.at[pl.ds(49920, 80)],
                            acc.at[pl.ds(49920, 80)])

    plsc.subcore_barrier()

    nchunks = jnp.where(s < _TOT_CHUNKS % 16,
                        _TOT_CHUNKS // 16 + 1, _TOT_CHUNKS // 16)
    lanes = lax.iota(jnp.int32, 16)

    def issue_inputs(t, bb):
        base = (s + t * 16) * _CHUNK
        pltpu.async_copy(edges.at[pl.ds(base * EDGE_T, _CHUNK * EDGE_T)],
                         ebufs[bb], sem_in)
        pltpu.async_copy(asrc.at[pl.ds(base, _CHUNK)],
                         abufs[bb].at[pl.ds(0, _CHUNK)], sem_in)

        @pl.when(c == 0)
        def _():
            pltpu.async_copy(xplo.at[pl.ds(base, _CHUNK)], xbufs[bb], sem_in)

        @pl.when(c == 1)
        def _():
            pltpu.async_copy(xphi.at[pl.ds(base, _CHUNK)], xbufs[bb], sem_in)

    def wait_inputs(bb):
        pltpu.make_async_copy(edges.at[pl.ds(0, _CHUNK * EDGE_T)],
                              ebufs[bb], sem_in).wait()
        pltpu.make_async_copy(asrc.at[pl.ds(0, _CHUNK)],
                              abufs[bb].at[pl.ds(0, _CHUNK)], sem_in).wait()
        pltpu.make_async_copy(xplo.at[pl.ds(0, _CHUNK)],
                              xbufs[bb], sem_in).wait()

    def issue_gathers(bb):
        pltpu.async_copy(adst.at[ebufs[bb].at[pl.ds(0, 128)]],
                         advAr[bb], semg0)
        pltpu.async_copy(adst.at[ebufs[bb].at[pl.ds(128, 128)]],
                         advBr[bb], semg1)

    def wait_gathers(bb, which):
        if which == 0:
            pltpu.make_async_copy(adst.at[ebufs[bb].at[pl.ds(0, 128)]],
                                  advAr[bb], semg0).wait()
        else:
            pltpu.make_async_copy(adst.at[ebufs[bb].at[pl.ds(128, 128)]],
                                  advBr[bb], semg1).wait()

    issue_inputs(0, 0)
    wait_inputs(0)
    issue_gathers(0)

    def pair_body(tt, carry):
        for bb in range(2):
            t = tt * 2 + bb

            @pl.when(t < nchunks)
            def _process(t=t, bb=bb):
                base = (s + t * 16) * _CHUNK

                @pl.when(t + 1 < nchunks)
                def _():
                    issue_inputs(t + 1, 1 - bb)

                hsc = [None] * 16
                for g in range(16):
                    adv = advAr[bb] if g < 8 else advBr[bb]
                    if g == 0:
                        wait_gathers(bb, 0)
                    if g == 8:
                        wait_gathers(bb, 1)
                        @pl.when(t + 1 < nchunks)
                        def _():
                            wait_inputs(1 - bb)
                            issue_gathers(1 - bb)
                    pg, pp = pays[g % 4], pixs[g % 4]
                    for k in range(1):
                        rr = g + k
                        ev = ebufs[bb][pl.ds(rr * 16, 16)]
                        ad = adv[pl.ds((rr * 16) % 128, 16)]
                        a = abufs[bb][pl.ds(rr, 16)][0] + ad
                        a = jnp.where(a >= 0, a, 0.2 * a)
                        ex = jnp.where(ev == base + rr, 0.0, jnp.exp(a))
                        pp[pl.ds(k * 16, 16)] = ev
                        x0 = xbufs[bb][rr, pl.ds(0, 16)]
                        x1 = xbufs[bb][rr, pl.ds(16, 16)]
                        for e in range(16):
                            se = ex[e]
                            row = k * 16 + e
                            pg[row, pl.ds(0, 16)] = jnp.where(lanes == 0,
                                                              se, 0.0)
                            pg[row, pl.ds(_NUM0, 16)] = se * x0
                            pg[row, pl.ds(_NUM0 + 16, 16)] = se * x1
                    if g < 3:
                        @pl.when(t > 0)
                        def _(g=g):
                            pltpu.make_async_copy(pays[g + 1],
                                                  acc.at[pixs[g + 1]],
                                                  sem_sc).wait()
                    else:
                        hsc[g - 3].wait()
                    hsc[g] = pltpu.async_copy(pg, acc.at[pp], sem_sc,
                                              add=True)

        return carry

    lax.fori_loop(0, _NPAIRS, pair_body, 0)
    for d in (1, 2, 3):
        pltpu.make_async_copy(pays[d], acc.at[pixs[d]], sem_sc).wait()
    plsc.subcore_barrier()

    j0 = s * _PT
    pltpu.sync_copy(bidx.at[pl.ds(j0, _PT)], bidx_v)
    for p in range(_PT // 16):
        pltpu.async_copy(acc.at[bidx_v.at[pl.ds(p * 16, 16)]],
                         pay0, sem_in).wait()
        pltpu.sync_copy(pay0, gat_o.at[c, pl.ds(j0 + p * 16, 16)])

    for p in range(_PTA // 16):
        a0 = c * (_B3 // 2) + s * _PTA + p * 16
        pltpu.sync_copy(bidx.at[pl.ds(a0, 16)], aeidx)
        pltpu.async_copy(ae.at[aeidx], aebuf, sem_in).wait()
        pltpu.sync_copy(aebuf, ae_o.at[pl.ds(a0, 16)])


def _sc_phase(edges_flat, a_src, a_dst, xp_lo, xp_hi, init0, init1, bidx,
              all_embed):
    mesh = plsc.VectorSubcoreMesh(core_axis_name="c", subcore_axis_name="s")
    f = pl.kernel(
        _sc_body,
        mesh=mesh,
        compiler_params=pltpu.CompilerParams(use_tc_tiling_on_sc=False),
        out_type=[
            jax.ShapeDtypeStruct((2, _B3, _ACC_W), jnp.float32),
            jax.ShapeDtypeStruct((_B3, EMB), jnp.float32),
        ],
        scratch_types=[
            pltpu.VMEM_SHARED((N_NODES, _ACC_W), jnp.float32),
            pltpu.VMEM((_CHUNK * EDGE_T,), jnp.int32),
            pltpu.VMEM((_CHUNK * EDGE_T,), jnp.int32),
            pltpu.VMEM((_CHUNK, HEMB), jnp.float32),
            pltpu.VMEM((_CHUNK, HEMB), jnp.float32),
            pltpu.VMEM((_CHUNK + 16,), jnp.float32),
            pltpu.VMEM((_CHUNK + 16,), jnp.float32),
            pltpu.VMEM((128,), jnp.float32),
            pltpu.VMEM((128,), jnp.float32),
            pltpu.VMEM((128,), jnp.float32),
            pltpu.VMEM((128,), jnp.float32),
            pltpu.VMEM((16, _ACC_W), jnp.float32),
            pltpu.VMEM((16, _ACC_W), jnp.float32),
            pltpu.VMEM((16, _ACC_W), jnp.float32),
            pltpu.VMEM((16, _ACC_W), jnp.float32),
            pltpu.VMEM((16,), jnp.int32),
            pltpu.VMEM((16,), jnp.int32),
            pltpu.VMEM((16,), jnp.int32),
            pltpu.VMEM((16,), jnp.int32),
            pltpu.VMEM((_PT,), jnp.int32),
            pltpu.VMEM((16,), jnp.int32),
            pltpu.VMEM((16, EMB), jnp.float32),
            pltpu.SemaphoreType.DMA,
            pltpu.SemaphoreType.DMA,
            pltpu.SemaphoreType.DMA,
            pltpu.SemaphoreType.DMA,
        ],
    )
    return f(edges_flat, a_src, a_dst, xp_lo, xp_hi, init0, init1, bidx,
             all_embed)


def _loss_body(gat, aer, b, loss_o, bpr_o, reg_o):
    gt = gat[...]
    den = gt[0, :, 0:1]
    num = jnp.concatenate([gt[0, :, _NUM0:_NUM0 + HEMB],
                           gt[1, :, _NUM0:_NUM0 + HEMB]], axis=1)
    gr = num / (den + 1e-16) + b[...]
    nrm = jnp.sqrt(jnp.sum(gr * gr, axis=1, keepdims=True))
    gr = gr / jnp.clip(nrm, 1e-12, None)
    aev = aer[...]
    aeu, aep, aen = aev[:BATCH], aev[BATCH:2 * BATCH], aev[2 * BATCH:]
    gu, gp, gn = gr[:BATCH], gr[BATCH:2 * BATCH], gr[2 * BATCH:]
    ps = jnp.sum(aeu * aep + gu * gp, axis=1)
    ns = jnp.sum(aeu * aen + gu * gn, axis=1)
    bpr = -jnp.mean(jnp.log(jax.nn.sigmoid(ps - ns)))
    reg = REGS * (jnp.sum(aeu ** 2) + jnp.sum(gu ** 2)
                  + jnp.sum(aep ** 2) + jnp.sum(gp ** 2)
                  + jnp.sum(aen ** 2) + jnp.sum(gn ** 2)) * 0.5
    loss_o[...] = (bpr + reg).reshape(1, 1)
    bpr_o[...] = bpr.reshape(1, 1)
    reg_o[...] = reg.reshape(1, 1)


def _loss_phase(gat, ae_rows, bias):
    loss, bpr, reg = pl.pallas_call(
        _loss_body,
        in_specs=[
            pl.BlockSpec((2, _B3, _ACC_W), lambda: (0, 0, 0)),
            pl.BlockSpec((_B3, EMB), lambda: (0, 0)),
            pl.BlockSpec((1, EMB), lambda: (0, 0)),
        ],
        out_specs=[
            pl.BlockSpec((1, 1), lambda: (0, 0)),
            pl.BlockSpec((1, 1), lambda: (0, 0)),
            pl.BlockSpec((1, 1), lambda: (0, 0)),
        ],
        out_shape=[jax.ShapeDtypeStruct((1, 1), jnp.float32)] * 3,
    )(gat, ae_rows, bias.reshape(1, EMB))
    return loss.reshape(()), bpr.reshape(()), reg.reshape(())


def kernel(user, pos_item, neg_item, edges_matrix, all_embed, W, att_src, att_dst, bias):
    xp_lo, xp_hi, a_src, a_dst, init0, init1 = _dense_phase(
        all_embed, W, att_src, att_dst)
    bidx = jnp.concatenate([user, pos_item, neg_item]).astype(jnp.int32)
    edges_flat = edges_matrix.reshape(-1)
    gat, ae_rows = _sc_phase(edges_flat, a_src.reshape(-1), a_dst.reshape(-1),
                             xp_lo, xp_hi, init0, init1, bidx, all_embed)
    loss, bpr, reg = _loss_phase(gat, ae_rows, bias)
    return (loss, bpr, reg)
